# Initial kernel scaffold; baseline (speedup 1.0000x reference)
import jax
import jax.numpy as jnp
from jax.experimental import pallas as pl


def kernel(jloc, joff, k):
    raise NotImplementedError("write your pallas kernel here")


# speedup vs baseline: 1.0864x; 1.0864x over previous
import jax
import jax.numpy as jnp
from jax.experimental import pallas as pl
from jax.experimental.pallas import tpu as pltpu


def _nms_body(jloc_ref, out_ref):
    x = jloc_ref[0]
    H, W = x.shape
    neginf = jnp.float32(-jnp.inf)
    up = jnp.concatenate([x[1:], jnp.full((1, W), neginf, x.dtype)], axis=0)
    dn = jnp.concatenate([jnp.full((1, W), neginf, x.dtype), x[:-1]], axis=0)
    m = jnp.maximum(jnp.maximum(x, up), dn)
    lf = jnp.concatenate([m[:, 1:], jnp.full((H, 1), neginf, x.dtype)], axis=1)
    rt = jnp.concatenate([jnp.full((H, 1), neginf, x.dtype), m[:, :-1]], axis=1)
    ap = jnp.maximum(jnp.maximum(m, lf), rt)
    out_ref[0] = jnp.where(x == ap, x, jnp.float32(0.0))


def kernel(jloc, joff, k):
    a = pl.pallas_call(
        _nms_body,
        out_shape=jax.ShapeDtypeStruct(jloc.shape, jloc.dtype),
    )(jloc)
    height, width = jloc.shape[1], jloc.shape[2]
    jloc_flat = a.reshape(-1)
    joff_flat = joff.reshape(2, -1)
    k_static = 20000
    scores, index = jax.lax.top_k(jloc_flat, k_static)
    scores = scores + (jnp.asarray(k) - k_static).astype(scores.dtype)
    y = (index // width).astype(jnp.float32) + jnp.take(joff_flat[1], index, axis=0) + 0.5
    x = (index % width).astype(jnp.float32) + jnp.take(joff_flat[0], index, axis=0) + 0.5
    junctions = jnp.stack((x, y)).T
    return jnp.concatenate([junctions, scores[:, None]], axis=1)


# speedup vs baseline: 1.9307x; 1.7772x over previous
import functools

import jax
import jax.numpy as jnp
from jax import lax
from jax.experimental import pallas as pl
from jax.experimental.pallas import tpu as pltpu
from jax.experimental.pallas import tpu_sc as plsc

H = W = 512
N = H * W
NT = 16
CH = N // NT
CHV = CH // 16
K = 20000
KPAD = 20480
SLICE = KPAD // NT
SLV = SLICE // 16
TRASH = KPAD
M = KPAD + 2048
MAGIC = 0x51C0000
PARK = MAGIC + 99
ASHIFT = 18
NB = 1 << (30 - ASHIFT)
NBS = NB // NT
LOWM = (1 << ASHIFT) - 1

TAGS = 0
HIST = 256
AK = HIST + NT * NB
AI = AK + M
BK = AI + M
BI = BK + M
SP_WORDS = BI + M

E_A1 = 1
E_A2 = 2
E_A3 = 3
E_B0 = 4
E_C1 = 16
E_C2 = 17
E_S0 = 18
E_F = 24
E_LAST = E_F


def _nms_body(jloc_ref, out_ref):
    x = jloc_ref[0]
    neginf = jnp.float32(-jnp.inf)
    up = jnp.concatenate([x[1:], jnp.full((1, W), neginf, x.dtype)], axis=0)
    dn = jnp.concatenate([jnp.full((1, W), neginf, x.dtype), x[:-1]], axis=0)
    m = jnp.maximum(jnp.maximum(x, up), dn)
    lf = jnp.concatenate([m[:, 1:], jnp.full((H, 1), neginf, x.dtype)], axis=1)
    rt = jnp.concatenate([jnp.full((H, 1), neginf, x.dtype), m[:, :-1]], axis=1)
    ap = jnp.maximum(jnp.maximum(m, lf), rt)
    a = jnp.where(x == ap, x, jnp.float32(0.0))
    out_ref[...] = lax.bitcast_convert_type(a, jnp.int32)


def _splat(v):
    return jnp.full((16,), v, jnp.int32)


def _sc_topk_kernel():
    mesh = plsc.VectorSubcoreMesh(core_axis_name="c", subcore_axis_name="s")

    @functools.partial(
        pl.kernel, mesh=mesh,
        out_type=[jax.ShapeDtypeStruct((KPAD,), jnp.float32),
                  jax.ShapeDtypeStruct((KPAD,), jnp.float32),
                  jax.ShapeDtypeStruct((KPAD,), jnp.float32)],
        scratch_types=[
            pltpu.VMEM((CH,), jnp.int32),
            pltpu.VMEM((CH,), jnp.int32),
            pltpu.VMEM((CH,), jnp.int32),
            pltpu.VMEM((16384,), jnp.int32),
            pltpu.VMEM((NBS,), jnp.int32),
            pltpu.VMEM((NBS,), jnp.int32),
            pltpu.VMEM((256,), jnp.int32),
            pltpu.VMEM((16,), jnp.int32),
            pltpu.VMEM((32 * 16,), jnp.int32),
            pltpu.VMEM((3 * 1024,), jnp.int32),
            pltpu.VMEM((1024,), jnp.int32),
            pltpu.VMEM((1024,), jnp.int32),
            pltpu.VMEM((SLICE,), jnp.int32),
            pltpu.VMEM((SLICE,), jnp.int32),
            pltpu.VMEM((SLICE,), jnp.int32),
            pltpu.VMEM((SLICE,), jnp.int32),
            pltpu.VMEM((SLICE,), jnp.float32),
            pltpu.VMEM((SLICE,), jnp.float32),
            pltpu.VMEM((SLICE,), jnp.float32),
            pltpu.VMEM_SHARED((SP_WORDS,), jnp.int32),
            pltpu.SemaphoreType.DMA,
        ],
        compiler_params=pltpu.CompilerParams(needs_layout_passes=False),
    )
    def sc_kernel(keys_hbm, joffx_hbm, joffy_hbm, outx_hbm, outy_hbm,
                  outs_hbm, v_chunk, v_gidx, v_pos, v_hist, v_g, v_g2,
                  v_land, v_h8, v_tagbank, v_shbank, v_bases, v_shist,
                  v_skey, v_sidx, v_spos, v_spos2, v_jx, v_jy, v_os,
                  sp_all, sem):
        cid = lax.axis_index("c")
        t = lax.axis_index("s")

        def i16():
            return lax.iota(jnp.int32, 16)

        def publish(e, myrow):
            row = jnp.where(i16() == 0, MAGIC + e, myrow)
            v_tagbank[pl.ds(e * 16, 16)] = row
            for tt in range(NT):
                @pl.when(t == tt)
                def _(tt=tt, e=e):
                    pltpu.sync_copy(v_tagbank.at[pl.ds(e * 16, 16)],
                                    sp_all.at[pl.ds(TAGS + tt * 16, 16)])
            return row

        def spin(e, last=False):
            def cond(carry):
                notdone, it = carry
                return jnp.logical_and(notdone, it < 300000)

            def body(carry):
                _, it = carry
                pltpu.sync_copy(sp_all.at[pl.ds(TAGS, 256)], v_land)
                col = plsc.load_gather(v_land, [i16() * 16])
                if last:
                    ok_lane = col >= MAGIC + e
                else:
                    ok_lane = jnp.logical_and(col >= MAGIC + e,
                                              col <= MAGIC + 64)
                nok = jnp.sum(ok_lane.astype(jnp.int32), axis=0)
                return nok < 16, it + 1

            lax.while_loop(cond, body, (jnp.bool_(True), jnp.int32(0)))

        def sync(e, myrow, last=False):
            row = publish(e, myrow)
            spin(e, last=last)
            return row

        def colv(c):
            return plsc.load_gather(v_land, [i16() * 16 + c])

        @pl.when(cid == 0)
        def _core0():
            ones = _splat(1)
            zeros = _splat(0)
            myrow = zeros

            pltpu.sync_copy(keys_hbm.at[pl.ds(t * CH, CH)], v_chunk)

            def zh(i, _):
                off = pl.multiple_of(i * 16, 16)
                v_hist[pl.ds(off, 16)] = zeros
                return _
            lax.fori_loop(0, NB // 16, zh, 0)

            def ah(i, _):
                off = pl.multiple_of(i * 16, 16)
                key = v_chunk[pl.ds(off, 16)]
                d = lax.shift_right_logical(key, ASHIFT)
                plsc.addupdate_scatter(v_hist, [d], ones)
                return _
            lax.fori_loop(0, CHV, ah, 0)
            for tt in range(NT):
                @pl.when(t == tt)
                def _(tt=tt):
                    pltpu.sync_copy(v_hist.at[pl.ds(0, NB)],
                                    sp_all.at[pl.ds(HIST + tt * NB, NB)])
            myrow = sync(E_A1, myrow)

            def zg(i, _):
                off = pl.multiple_of(i * 16, 16)
                v_g[pl.ds(off, 16)] = zeros
                return _
            lax.fori_loop(0, NBS // 16, zg, 0)

            for rr in range(NT):
                pltpu.sync_copy(
                    sp_all.at[pl.ds(HIST + rr * NB + t * NBS, NBS)], v_g2)

                def addv(i, __):
                    off = pl.multiple_of(i * 16, 16)
                    v_g[pl.ds(off, 16)] = (v_g[pl.ds(off, 16)]
                                           + v_g2[pl.ds(off, 16)])
                    return __
                lax.fori_loop(0, NBS // 16, addv, 0)

            def tot_acc(i, acc):
                off = pl.multiple_of(i * 16, 16)
                return acc + v_g[pl.ds(off, 16)]
            tot_vec = lax.fori_loop(0, NBS // 16, tot_acc, zeros)
            tot_slice = jnp.sum(tot_vec, axis=0)
            myrow = jnp.where(i16() == 12, tot_slice, myrow)
            myrow = sync(E_A2, myrow)

            tots = colv(12)
            sfx = lax.rev(plsc.cumsum(lax.rev(tots, (0,))), (0,))
            s_above_v = sfx - tots
            s_above = jnp.sum(jnp.where(i16() == t, s_above_v, 0), axis=0)

            def find(iv, carry):
                found, bstar, n_hi, running = carry
                v = NBS // 16 - 1 - iv
                off = pl.multiple_of(v * 16, 16)
                c = v_g[pl.ds(off, 16)]
                sfx_in = lax.rev(plsc.cumsum(lax.rev(c, (0,))), (0,))
                s_excl = running + sfx_in - c
                cond = jnp.logical_and(s_excl < K, K <= s_excl + c)
                anyc = jnp.sum(cond.astype(jnp.int32), axis=0) > 0
                d_here = jnp.sum(jnp.where(cond, t * NBS + off + i16(), 0),
                                 axis=0)
                nh_here = jnp.sum(jnp.where(cond, s_excl, 0), axis=0)
                hit = jnp.logical_and(anyc, jnp.logical_not(found))
                bstar = jnp.where(hit, d_here, bstar)
                n_hi = jnp.where(hit, nh_here, n_hi)
                found = jnp.logical_or(found, anyc)
                running = running + jnp.sum(c, axis=0)
                return found, bstar, n_hi, running
            found, bstar, n_hi, _ = lax.fori_loop(
                0, NBS // 16, find,
                (jnp.bool_(False), jnp.int32(0), jnp.int32(0), s_above))

            myrow = jnp.where(i16() == 9, found.astype(jnp.int32), myrow)
            myrow = jnp.where(i16() == 10, bstar, myrow)
            myrow = jnp.where(i16() == 11, n_hi, myrow)
            myrow = sync(E_A3, myrow)

            fcol = colv(9)
            bstar = jnp.sum(fcol * colv(10), axis=0)
            n_hi = jnp.sum(fcol * colv(11), axis=0)

            def compact(i, cnt):
                off = pl.multiple_of(i * 16, 16)
                key = v_chunk[pl.ds(off, 16)]
                d = lax.shift_right_logical(key, ASHIFT)
                m = d == bstar
                mi = m.astype(jnp.int32)
                ex = plsc.cumsum(mi) - mi
                pos = cnt + ex
                plsc.store_scatter(v_pos, [pos],
                                   jnp.bitwise_and(key, LOWM), mask=m)
                return cnt + jnp.sum(mi, axis=0)
            neq_list = lax.fori_loop(0, CHV, compact, jnp.int32(0))
            nv_eq = lax.div(neq_list + 15, 16)

            n_above = n_hi
            prefix = jnp.int32(0)
            for r in range(6):
                sh = 15 - 3 * r
                v_h8[...] = zeros

                def hrow(i, _, sh=sh, prefix=prefix):
                    off = pl.multiple_of(i * 16, 16)
                    lw = v_pos[pl.ds(off, 16)]
                    valid = off + i16() < neq_list
                    pref_ok = lax.shift_right_logical(lw, sh + 3) == prefix
                    m = jnp.logical_and(valid, pref_ok)
                    b = jnp.bitwise_and(lax.shift_right_logical(lw, sh), 7)
                    plsc.addupdate_scatter(v_h8, [b], ones, mask=m)
                    return _
                lax.fori_loop(0, nv_eq, hrow, 0)
                h8 = v_h8[...]
                g = plsc.load_gather(v_h8, [jnp.clip(i16() - 1, 0, 15)])
                incols = jnp.logical_and(i16() >= 1, i16() <= 8)
                myrow = jnp.where(incols, g, myrow)
                myrow = sync(E_B0 + 2 * r, myrow)

                gcnt = zeros
                for c in range(8):
                    s = jnp.sum(colv(1 + c), axis=0)
                    gcnt = gcnt + jnp.where(i16() == c, s, 0)
                sfx8 = lax.rev(plsc.cumsum(lax.rev(gcnt, (0,))), (0,))
                ex8 = n_above + sfx8 - gcnt
                cond8 = jnp.logical_and(ex8 < K, K <= ex8 + gcnt)
                beta = jnp.sum(jnp.where(cond8, i16(), 0), axis=0)
                n_above = jnp.sum(jnp.where(cond8, ex8, 0), axis=0)
                prefix = prefix * 8 + beta
                myrow = sync(E_B0 + 2 * r + 1, myrow)

            lstar = prefix
            n_gt = n_above
            T = jnp.bitwise_or(lax.shift_left(bstar, ASHIFT), lstar)

            def c1(i, carry):
                ngt, neq = carry
                off = pl.multiple_of(i * 16, 16)
                key = v_chunk[pl.ds(off, 16)]
                ngt = ngt + jnp.sum((key > T).astype(jnp.int32), axis=0)
                neq = neq + jnp.sum((key == T).astype(jnp.int32), axis=0)
                return ngt, neq
            ngt_t, neq_t = lax.fori_loop(0, CHV, c1,
                                         (jnp.int32(0), jnp.int32(0)))
            myrow = jnp.where(i16() == 13, ngt_t, myrow)
            myrow = jnp.where(i16() == 14, neq_t, myrow)
            myrow = sync(E_C1, myrow)

            ngts = colv(13)
            neqs = colv(14)
            pre_gt = plsc.cumsum(ngts) - ngts
            pre_eq = plsc.cumsum(neqs) - neqs
            sel = i16() == t
            base_gt = jnp.sum(jnp.where(sel, pre_gt, 0), axis=0)
            eqpre = jnp.sum(jnp.where(sel, pre_eq, 0), axis=0)
            n_tie = K - n_gt

            def c2(i, carry):
                gtrun, eqrun = carry
                off = pl.multiple_of(i * 16, 16)
                key = v_chunk[pl.ds(off, 16)]
                m_gt = key > T
                m_eq = key == T
                gi = m_gt.astype(jnp.int32)
                ei = m_eq.astype(jnp.int32)
                ex_gt = plsc.cumsum(gi) - gi
                ex_eq = plsc.cumsum(ei) - ei
                pos_gt = gtrun + ex_gt
                eq_rank = eqrun + ex_eq
                keep_eq = jnp.logical_and(m_eq, eq_rank < n_tie)
                pos_eq = n_gt + eq_rank
                trash = TRASH + jnp.bitwise_and(off, 2047) + i16()
                pos = jnp.where(m_gt, pos_gt,
                                jnp.where(keep_eq, pos_eq, trash))
                pos = jnp.clip(pos, 0, M - 1)
                v_pos[pl.ds(off, 16)] = AK + pos
                v_hist[pl.ds(off, 16)] = AI + pos
                v_chunk[pl.ds(off, 16)] = key - T
                v_gidx[pl.ds(off, 16)] = t * CH + off + i16()
                return (gtrun + jnp.sum(gi, axis=0),
                        eqrun + jnp.sum(ei, axis=0))
            lax.fori_loop(0, CHV, c2, (base_gt, eqpre))
            pltpu.sync_copy(v_chunk, sp_all.at[v_pos])
            pltpu.sync_copy(v_gidx, sp_all.at[v_hist])

            @pl.when(t == NT - 1)
            def _pads():
                def zp(i, _):
                    off = pl.multiple_of(i * 16, 16)
                    v_skey[pl.ds(off, 16)] = zeros
                    return _
                lax.fori_loop(0, (KPAD - K) // 16, zp, 0)
                pltpu.sync_copy(v_skey.at[pl.ds(0, KPAD - K)],
                                sp_all.at[pl.ds(AK + K, KPAD - K)])
                pltpu.sync_copy(v_skey.at[pl.ds(0, KPAD - K)],
                                sp_all.at[pl.ds(AI + K, KPAD - K)])
            myrow = sync(E_C2, myrow)

            def sort_pass(p, src_k, src_i, dst_k, dst_i, e, myrow):
                sb = pl.multiple_of(t * SLICE, SLICE)
                pltpu.sync_copy(sp_all.at[pl.ds(src_k + sb, SLICE)], v_skey)
                pltpu.sync_copy(sp_all.at[pl.ds(src_i + sb, SLICE)], v_sidx)

                hrow_ref = v_shbank.at[pl.ds(p * 1024, 1024)]

                def zsh(i, _):
                    off = pl.multiple_of(i * 16, 16)
                    hrow_ref[pl.ds(off, 16)] = zeros
                    return _
                lax.fori_loop(0, 64, zsh, 0)

                def shl(i, _):
                    off = pl.multiple_of(i * 16, 16)
                    kk = v_skey[pl.ds(off, 16)]
                    d = jnp.bitwise_and(
                        lax.shift_right_logical(kk, 10 * p), 1023)
                    plsc.addupdate_scatter(hrow_ref, [d], ones)
                    return _
                lax.fori_loop(0, SLV, shl, 0)
                for tt in range(NT):
                    @pl.when(t == tt)
                    def _(tt=tt):
                        pltpu.sync_copy(
                            hrow_ref,
                            sp_all.at[pl.ds(HIST + tt * NB, 1024)])
                myrow = sync(e, myrow)

                for rr in range(NT):
                    pltpu.sync_copy(sp_all.at[pl.ds(HIST + rr * NB, 1024)],
                                    v_hist.at[pl.ds(rr * 1024, 1024)])

                def dig2(i, _):
                    off = pl.multiple_of(i * 16, 16)

                    def rows(r, carry):
                        tot, pre = carry
                        roff = pl.multiple_of(r * 1024, 1024)
                        c = v_hist[pl.ds(roff + off, 16)]
                        pre = pre + jnp.where(r < t, c, 0)
                        return tot + c, pre
                    tot, pre = lax.fori_loop(0, 16, rows, (zeros, zeros))
                    v_bases[pl.ds(off, 16)] = pre
                    v_shist[pl.ds(off, 16)] = tot
                    return _
                lax.fori_loop(0, 64, dig2, 0)

                def sfxl(iv, running):
                    v = 63 - iv
                    off = pl.multiple_of(v * 16, 16)
                    c = v_shist[pl.ds(off, 16)]
                    sfx_in = lax.rev(plsc.cumsum(lax.rev(c, (0,))), (0,))
                    s_excl = running + sfx_in - c
                    v_bases[pl.ds(off, 16)] = v_bases[pl.ds(off, 16)] + s_excl
                    return running + jnp.sum(c, axis=0)
                lax.fori_loop(0, 64, sfxl, jnp.int32(0))

                def scat(i, _):
                    off = pl.multiple_of(i * 16, 16)
                    kk = v_skey[pl.ds(off, 16)]
                    d = jnp.bitwise_and(
                        lax.shift_right_logical(kk, 10 * p), 1023)
                    occ, lastm = plsc.scan_count(d)
                    base_d = plsc.load_gather(v_bases, [d])
                    pp = jnp.clip(base_d + occ - 1, 0, M - 1)
                    v_spos[pl.ds(off, 16)] = dst_k + pp
                    v_spos2[pl.ds(off, 16)] = dst_i + pp
                    plsc.addupdate_scatter(v_bases, [d], occ, mask=lastm)
                    return _
                lax.fori_loop(0, SLV, scat, 0)
                pltpu.sync_copy(v_skey, sp_all.at[v_spos])
                pltpu.sync_copy(v_sidx, sp_all.at[v_spos2])
                return sync(e + 1, myrow)

            myrow = sort_pass(0, AK, AI, BK, BI, E_S0, myrow)
            myrow = sort_pass(1, BK, BI, AK, AI, E_S0 + 2, myrow)
            myrow = sort_pass(2, AK, AI, BK, BI, E_S0 + 4, myrow)

            sb = pl.multiple_of(t * SLICE, SLICE)
            pltpu.sync_copy(sp_all.at[pl.ds(BK + sb, SLICE)], v_skey)
            pltpu.sync_copy(sp_all.at[pl.ds(BI + sb, SLICE)], v_sidx)

            def clampi(i, _):
                off = pl.multiple_of(i * 16, 16)
                v_sidx[pl.ds(off, 16)] = jnp.clip(v_sidx[pl.ds(off, 16)],
                                                  0, N - 1)
                return _
            lax.fori_loop(0, SLV, clampi, 0)
            pltpu.async_copy(joffx_hbm.at[v_sidx], v_jx, sem).wait()
            pltpu.async_copy(joffy_hbm.at[v_sidx], v_jy, sem).wait()

            def emit(i, _):
                off = pl.multiple_of(i * 16, 16)
                kk = v_skey[pl.ds(off, 16)]
                idx = v_sidx[pl.ds(off, 16)]
                score = plsc.bitcast(kk + T, jnp.float32)
                rowf = lax.shift_right_logical(idx, 9).astype(jnp.float32)
                colf = jnp.bitwise_and(idx, W - 1).astype(jnp.float32)
                v_jx[pl.ds(off, 16)] = (colf + v_jx[pl.ds(off, 16)]) + 0.5
                v_jy[pl.ds(off, 16)] = (rowf + v_jy[pl.ds(off, 16)]) + 0.5
                v_os[pl.ds(off, 16)] = score
                return _
            lax.fori_loop(0, SLV, emit, 0)
            pltpu.sync_copy(v_jx, outx_hbm.at[pl.ds(sb, SLICE)])
            pltpu.sync_copy(v_jy, outy_hbm.at[pl.ds(sb, SLICE)])
            pltpu.sync_copy(v_os, outs_hbm.at[pl.ds(sb, SLICE)])

            myrow = sync(E_LAST, myrow, last=True)
            v_tagbank[pl.ds(31 * 16, 16)] = _splat(PARK)
            for tt in range(NT):
                @pl.when(t == tt)
                def _(tt=tt):
                    pltpu.sync_copy(v_tagbank.at[pl.ds(31 * 16, 16)],
                                    sp_all.at[pl.ds(TAGS + tt * 16, 16)])

    return sc_kernel


_SC_KERNEL = None


def kernel(jloc, joff, k):
    global _SC_KERNEL
    if _SC_KERNEL is None:
        _SC_KERNEL = _sc_topk_kernel()
    keys2d = pl.pallas_call(
        _nms_body,
        out_shape=jax.ShapeDtypeStruct((H, W), jnp.int32),
    )(jloc)
    keys = keys2d.reshape(-1)
    joff_flat = joff.reshape(2, -1)
    outx, outy, outs_ = _SC_KERNEL(keys, joff_flat[0], joff_flat[1])
    x = outx[:K]
    y = outy[:K]
    scores = outs_[:K] + (jnp.asarray(k) - K).astype(jnp.float32)
    junctions = jnp.stack((x, y)).T
    return jnp.concatenate([junctions, scores[:, None]], axis=1)


# speedup vs baseline: 1.9768x; 1.0239x over previous
import functools

import jax
import jax.numpy as jnp
from jax import lax
from jax.experimental import pallas as pl
from jax.experimental.pallas import tpu as pltpu
from jax.experimental.pallas import tpu_sc as plsc

H = W = 512
N = H * W
NT = 16
CH = N // NT
CHV = CH // 16
K = 20000
KPAD = 20480
SLICE = KPAD // NT
SLV = SLICE // 16
TRASH = KPAD
M = KPAD + 2048
MAGIC = 0x51C0000
PARK = MAGIC + 99
ASHIFT = 18
NB = 1 << (30 - ASHIFT)
NBS = NB // NT
LOWM = (1 << ASHIFT) - 1

TAGS = 0
HIST = 256
AK = HIST + NT * NB
AI = AK + M
BK = AI + M
BI = BK + M
SP_WORDS = BI + M

E_A1 = 1
E_A2 = 2
E_A3 = 3
E_B0 = 4
E_C1 = 16
E_C2 = 17
E_S0 = 18
E_F = 24
E_LAST = E_F


def _nms_body(jloc_ref, out_ref):
    x = jloc_ref[0]
    neginf = jnp.float32(-jnp.inf)
    up = jnp.concatenate([x[1:], jnp.full((1, W), neginf, x.dtype)], axis=0)
    dn = jnp.concatenate([jnp.full((1, W), neginf, x.dtype), x[:-1]], axis=0)
    m = jnp.maximum(jnp.maximum(x, up), dn)
    lf = jnp.concatenate([m[:, 1:], jnp.full((H, 1), neginf, x.dtype)], axis=1)
    rt = jnp.concatenate([jnp.full((H, 1), neginf, x.dtype), m[:, :-1]], axis=1)
    ap = jnp.maximum(jnp.maximum(m, lf), rt)
    a = jnp.where(x == ap, x, jnp.float32(0.0))
    out_ref[...] = lax.bitcast_convert_type(a, jnp.int32)


def _splat(v):
    return jnp.full((16,), v, jnp.int32)


def _sc_topk_kernel():
    mesh = plsc.VectorSubcoreMesh(core_axis_name="c", subcore_axis_name="s")

    @functools.partial(
        pl.kernel, mesh=mesh,
        out_type=[jax.ShapeDtypeStruct((KPAD,), jnp.float32),
                  jax.ShapeDtypeStruct((KPAD,), jnp.float32),
                  jax.ShapeDtypeStruct((KPAD,), jnp.float32)],
        scratch_types=[
            pltpu.VMEM((CH,), jnp.int32),
            pltpu.VMEM((CH,), jnp.int32),
            pltpu.VMEM((CH,), jnp.int32),
            pltpu.VMEM((16384,), jnp.int32),
            pltpu.VMEM((NBS,), jnp.int32),
            pltpu.VMEM((NBS,), jnp.int32),
            pltpu.VMEM((256,), jnp.int32),
            pltpu.VMEM((16,), jnp.int32),
            pltpu.VMEM((32 * 16,), jnp.int32),
            pltpu.VMEM((3 * 1024,), jnp.int32),
            pltpu.VMEM((1024,), jnp.int32),
            pltpu.VMEM((1024,), jnp.int32),
            pltpu.VMEM((SLICE,), jnp.int32),
            pltpu.VMEM((SLICE,), jnp.int32),
            pltpu.VMEM((SLICE,), jnp.int32),
            pltpu.VMEM((SLICE,), jnp.int32),
            pltpu.VMEM((SLICE,), jnp.float32),
            pltpu.VMEM((SLICE,), jnp.float32),
            pltpu.VMEM((SLICE,), jnp.float32),
            pltpu.VMEM_SHARED((SP_WORDS,), jnp.int32),
            pltpu.SemaphoreType.DMA,
        ],
        compiler_params=pltpu.CompilerParams(needs_layout_passes=False),
    )
    def sc_kernel(keys_hbm, joffx_hbm, joffy_hbm, outx_hbm, outy_hbm,
                  outs_hbm, v_chunk, v_gidx, v_pos, v_hist, v_g, v_g2,
                  v_land, v_h8, v_tagbank, v_shbank, v_bases, v_shist,
                  v_skey, v_sidx, v_spos, v_spos2, v_jx, v_jy, v_os,
                  sp_all, sem):
        cid = lax.axis_index("c")
        t = lax.axis_index("s")

        def i16():
            return lax.iota(jnp.int32, 16)

        def publish(e, myrow):
            row = jnp.where(i16() == 0, MAGIC + e, myrow)
            v_tagbank[pl.ds(e * 16, 16)] = row
            for tt in range(NT):
                @pl.when(t == tt)
                def _(tt=tt, e=e):
                    pltpu.sync_copy(v_tagbank.at[pl.ds(e * 16, 16)],
                                    sp_all.at[pl.ds(TAGS + tt * 16, 16)])
            return row

        def spin(e, last=False):
            def cond(carry):
                notdone, it = carry
                return jnp.logical_and(notdone, it < 300000)

            def body(carry):
                _, it = carry
                pltpu.sync_copy(sp_all.at[pl.ds(TAGS, 256)], v_land)
                col = plsc.load_gather(v_land, [i16() * 16])
                if last:
                    ok_lane = col >= MAGIC + e
                else:
                    ok_lane = jnp.logical_and(col >= MAGIC + e,
                                              col <= MAGIC + 64)
                nok = jnp.sum(ok_lane.astype(jnp.int32), axis=0)
                return nok < 16, it + 1

            lax.while_loop(cond, body, (jnp.bool_(True), jnp.int32(0)))

        def sync(e, myrow, last=False):
            row = publish(e, myrow)
            spin(e, last=last)
            return row

        def colv(c):
            return plsc.load_gather(v_land, [i16() * 16 + c])

        @pl.when(cid == 0)
        def _core0():
            ones = _splat(1)
            zeros = _splat(0)
            myrow = zeros

            pltpu.sync_copy(keys_hbm.at[pl.ds(t * CH, CH)], v_chunk)

            def zh(i, _):
                off = pl.multiple_of(i * 16, 16)
                v_hist[pl.ds(off, 16)] = zeros
                return _
            lax.fori_loop(0, NB // 16, zh, 0)

            def ah(i, mx):
                off = pl.multiple_of(i * 16, 16)
                key = v_chunk[pl.ds(off, 16)]
                d = lax.shift_right_logical(key, ASHIFT)
                plsc.addupdate_scatter(v_hist, [d], ones)
                return jnp.maximum(mx, key)
            maxvec = lax.fori_loop(0, CHV, ah, zeros)
            maxkey_t = jnp.max(maxvec, axis=0)
            for tt in range(NT):
                @pl.when(t == tt)
                def _(tt=tt):
                    pltpu.sync_copy(v_hist.at[pl.ds(0, NB)],
                                    sp_all.at[pl.ds(HIST + tt * NB, NB)])
            myrow = sync(E_A1, myrow)

            def zg(i, _):
                off = pl.multiple_of(i * 16, 16)
                v_g[pl.ds(off, 16)] = zeros
                return _
            lax.fori_loop(0, NBS // 16, zg, 0)

            for rr in range(NT):
                pltpu.sync_copy(
                    sp_all.at[pl.ds(HIST + rr * NB + t * NBS, NBS)], v_g2)

                def addv(i, __):
                    off = pl.multiple_of(i * 16, 16)
                    v_g[pl.ds(off, 16)] = (v_g[pl.ds(off, 16)]
                                           + v_g2[pl.ds(off, 16)])
                    return __
                lax.fori_loop(0, NBS // 16, addv, 0)

            def tot_acc(i, acc):
                off = pl.multiple_of(i * 16, 16)
                return acc + v_g[pl.ds(off, 16)]
            tot_vec = lax.fori_loop(0, NBS // 16, tot_acc, zeros)
            tot_slice = jnp.sum(tot_vec, axis=0)
            myrow = jnp.where(i16() == 12, tot_slice, myrow)
            myrow = jnp.where(i16() == 15, maxkey_t, myrow)
            myrow = sync(E_A2, myrow)

            maxkey = jnp.max(colv(15), axis=0)
            tots = colv(12)
            sfx = lax.rev(plsc.cumsum(lax.rev(tots, (0,))), (0,))
            s_above_v = sfx - tots
            s_above = jnp.sum(jnp.where(i16() == t, s_above_v, 0), axis=0)

            def find(iv, carry):
                found, bstar, n_hi, running = carry
                v = NBS // 16 - 1 - iv
                off = pl.multiple_of(v * 16, 16)
                c = v_g[pl.ds(off, 16)]
                sfx_in = lax.rev(plsc.cumsum(lax.rev(c, (0,))), (0,))
                s_excl = running + sfx_in - c
                cond = jnp.logical_and(s_excl < K, K <= s_excl + c)
                anyc = jnp.sum(cond.astype(jnp.int32), axis=0) > 0
                d_here = jnp.sum(jnp.where(cond, t * NBS + off + i16(), 0),
                                 axis=0)
                nh_here = jnp.sum(jnp.where(cond, s_excl, 0), axis=0)
                hit = jnp.logical_and(anyc, jnp.logical_not(found))
                bstar = jnp.where(hit, d_here, bstar)
                n_hi = jnp.where(hit, nh_here, n_hi)
                found = jnp.logical_or(found, anyc)
                running = running + jnp.sum(c, axis=0)
                return found, bstar, n_hi, running
            found, bstar, n_hi, _ = lax.fori_loop(
                0, NBS // 16, find,
                (jnp.bool_(False), jnp.int32(0), jnp.int32(0), s_above))

            myrow = jnp.where(i16() == 9, found.astype(jnp.int32), myrow)
            myrow = jnp.where(i16() == 10, bstar, myrow)
            myrow = jnp.where(i16() == 11, n_hi, myrow)
            myrow = sync(E_A3, myrow)

            fcol = colv(9)
            bstar = jnp.sum(fcol * colv(10), axis=0)
            n_hi = jnp.sum(fcol * colv(11), axis=0)

            def compact(i, carry):
                cnt, cgt = carry
                off = pl.multiple_of(i * 16, 16)
                key = v_chunk[pl.ds(off, 16)]
                d = lax.shift_right_logical(key, ASHIFT)
                m = d == bstar
                mi = m.astype(jnp.int32)
                ex = plsc.cumsum(mi) - mi
                pos = cnt + ex
                plsc.store_scatter(v_pos, [pos],
                                   jnp.bitwise_and(key, LOWM), mask=m)
                cgt = cgt + jnp.sum((d > bstar).astype(jnp.int32), axis=0)
                return cnt + jnp.sum(mi, axis=0), cgt
            neq_list, cgt_t = lax.fori_loop(0, CHV, compact,
                                            (jnp.int32(0), jnp.int32(0)))
            nv_eq = lax.div(neq_list + 15, 16)

            n_above = n_hi
            prefix = jnp.int32(0)
            for r in range(6):
                sh = 15 - 3 * r
                v_h8[...] = zeros

                def hrow(i, _, sh=sh, prefix=prefix):
                    off = pl.multiple_of(i * 16, 16)
                    lw = v_pos[pl.ds(off, 16)]
                    valid = off + i16() < neq_list
                    pref_ok = lax.shift_right_logical(lw, sh + 3) == prefix
                    m = jnp.logical_and(valid, pref_ok)
                    b = jnp.bitwise_and(lax.shift_right_logical(lw, sh), 7)
                    plsc.addupdate_scatter(v_h8, [b], ones, mask=m)
                    return _
                lax.fori_loop(0, nv_eq, hrow, 0)
                h8 = v_h8[...]
                g = plsc.load_gather(v_h8, [jnp.clip(i16() - 1, 0, 15)])
                incols = jnp.logical_and(i16() >= 1, i16() <= 8)
                myrow = jnp.where(incols, g, myrow)
                myrow = sync(E_B0 + 2 * r, myrow)

                gcnt = zeros
                for c in range(8):
                    s = jnp.sum(colv(1 + c), axis=0)
                    gcnt = gcnt + jnp.where(i16() == c, s, 0)
                sfx8 = lax.rev(plsc.cumsum(lax.rev(gcnt, (0,))), (0,))
                ex8 = n_above + sfx8 - gcnt
                cond8 = jnp.logical_and(ex8 < K, K <= ex8 + gcnt)
                beta = jnp.sum(jnp.where(cond8, i16(), 0), axis=0)
                n_above = jnp.sum(jnp.where(cond8, ex8, 0), axis=0)
                prefix = prefix * 8 + beta
                myrow = sync(E_B0 + 2 * r + 1, myrow)

            lstar = prefix
            n_gt = n_above
            T = jnp.bitwise_or(lax.shift_left(bstar, ASHIFT), lstar)

            def c1(i, carry):
                ngt, neq = carry
                off = pl.multiple_of(i * 16, 16)
                lw = v_pos[pl.ds(off, 16)]
                valid = off + i16() < neq_list
                gt = jnp.logical_and(valid, lw > lstar)
                eq = jnp.logical_and(valid, lw == lstar)
                ngt = ngt + jnp.sum(gt.astype(jnp.int32), axis=0)
                neq = neq + jnp.sum(eq.astype(jnp.int32), axis=0)
                return ngt, neq
            ngt_eq, neq_t = lax.fori_loop(0, nv_eq, c1,
                                          (jnp.int32(0), jnp.int32(0)))
            ngt_t = cgt_t + ngt_eq
            myrow = jnp.where(i16() == 13, ngt_t, myrow)
            myrow = jnp.where(i16() == 14, neq_t, myrow)
            myrow = sync(E_C1, myrow)

            ngts = colv(13)
            neqs = colv(14)
            pre_gt = plsc.cumsum(ngts) - ngts
            pre_eq = plsc.cumsum(neqs) - neqs
            sel = i16() == t
            base_gt = jnp.sum(jnp.where(sel, pre_gt, 0), axis=0)
            eqpre = jnp.sum(jnp.where(sel, pre_eq, 0), axis=0)
            n_tie = K - n_gt

            def c2(i, carry):
                gtrun, eqrun = carry
                off = pl.multiple_of(i * 16, 16)
                key = v_chunk[pl.ds(off, 16)]
                m_gt = key > T
                m_eq = key == T
                gi = m_gt.astype(jnp.int32)
                ei = m_eq.astype(jnp.int32)
                ex_gt = plsc.cumsum(gi) - gi
                ex_eq = plsc.cumsum(ei) - ei
                pos_gt = gtrun + ex_gt
                eq_rank = eqrun + ex_eq
                keep_eq = jnp.logical_and(m_eq, eq_rank < n_tie)
                pos_eq = n_gt + eq_rank
                trash = TRASH + jnp.bitwise_and(off, 2047) + i16()
                pos = jnp.where(m_gt, pos_gt,
                                jnp.where(keep_eq, pos_eq, trash))
                pos = jnp.clip(pos, 0, M - 1)
                v_pos[pl.ds(off, 16)] = AK + pos
                v_hist[pl.ds(off, 16)] = AI + pos
                v_chunk[pl.ds(off, 16)] = key - T
                v_gidx[pl.ds(off, 16)] = t * CH + off + i16()
                return (gtrun + jnp.sum(gi, axis=0),
                        eqrun + jnp.sum(ei, axis=0))
            lax.fori_loop(0, CHV, c2, (base_gt, eqpre))
            pltpu.sync_copy(v_chunk, sp_all.at[v_pos])
            pltpu.sync_copy(v_gidx, sp_all.at[v_hist])

            @pl.when(t == NT - 1)
            def _pads():
                def zp(i, _):
                    off = pl.multiple_of(i * 16, 16)
                    v_skey[pl.ds(off, 16)] = zeros
                    return _
                lax.fori_loop(0, (KPAD - K) // 16, zp, 0)
                pltpu.sync_copy(v_skey.at[pl.ds(0, KPAD - K)],
                                sp_all.at[pl.ds(AK + K, KPAD - K)])
                pltpu.sync_copy(v_skey.at[pl.ds(0, KPAD - K)],
                                sp_all.at[pl.ds(AI + K, KPAD - K)])
            myrow = sync(E_C2, myrow)

            def sort_pass(p, src_k, src_i, dst_k, dst_i, e, myrow):
                sb = pl.multiple_of(t * SLICE, SLICE)
                pltpu.sync_copy(sp_all.at[pl.ds(src_k + sb, SLICE)], v_skey)
                pltpu.sync_copy(sp_all.at[pl.ds(src_i + sb, SLICE)], v_sidx)

                hrow_ref = v_shbank.at[pl.ds(p * 1024, 1024)]

                def zsh(i, _):
                    off = pl.multiple_of(i * 16, 16)
                    hrow_ref[pl.ds(off, 16)] = zeros
                    return _
                lax.fori_loop(0, 64, zsh, 0)

                def shl(i, _):
                    off = pl.multiple_of(i * 16, 16)
                    kk = v_skey[pl.ds(off, 16)]
                    d = jnp.bitwise_and(
                        lax.shift_right_logical(kk, 10 * p), 1023)
                    plsc.addupdate_scatter(hrow_ref, [d], ones)
                    return _
                lax.fori_loop(0, SLV, shl, 0)
                for tt in range(NT):
                    @pl.when(t == tt)
                    def _(tt=tt):
                        pltpu.sync_copy(
                            hrow_ref,
                            sp_all.at[pl.ds(HIST + tt * NB, 1024)])
                myrow = sync(e, myrow)

                for rr in range(NT):
                    pltpu.sync_copy(sp_all.at[pl.ds(HIST + rr * NB, 1024)],
                                    v_hist.at[pl.ds(rr * 1024, 1024)])

                def dig2(i, _):
                    off = pl.multiple_of(i * 16, 16)

                    def rows(r, carry):
                        tot, pre = carry
                        roff = pl.multiple_of(r * 1024, 1024)
                        c = v_hist[pl.ds(roff + off, 16)]
                        pre = pre + jnp.where(r < t, c, 0)
                        return tot + c, pre
                    tot, pre = lax.fori_loop(0, 16, rows, (zeros, zeros))
                    v_bases[pl.ds(off, 16)] = pre
                    v_shist[pl.ds(off, 16)] = tot
                    return _
                lax.fori_loop(0, 64, dig2, 0)

                def sfxl(iv, running):
                    v = 63 - iv
                    off = pl.multiple_of(v * 16, 16)
                    c = v_shist[pl.ds(off, 16)]
                    sfx_in = lax.rev(plsc.cumsum(lax.rev(c, (0,))), (0,))
                    s_excl = running + sfx_in - c
                    v_bases[pl.ds(off, 16)] = v_bases[pl.ds(off, 16)] + s_excl
                    return running + jnp.sum(c, axis=0)
                lax.fori_loop(0, 64, sfxl, jnp.int32(0))

                def scat(i, _):
                    off = pl.multiple_of(i * 16, 16)
                    kk = v_skey[pl.ds(off, 16)]
                    d = jnp.bitwise_and(
                        lax.shift_right_logical(kk, 10 * p), 1023)
                    occ, lastm = plsc.scan_count(d)
                    base_d = plsc.load_gather(v_bases, [d])
                    pp = jnp.clip(base_d + occ - 1, 0, M - 1)
                    v_spos[pl.ds(off, 16)] = dst_k + pp
                    v_spos2[pl.ds(off, 16)] = dst_i + pp
                    plsc.addupdate_scatter(v_bases, [d], occ, mask=lastm)
                    return _
                lax.fori_loop(0, SLV, scat, 0)
                pltpu.sync_copy(v_skey, sp_all.at[v_spos])
                pltpu.sync_copy(v_sidx, sp_all.at[v_spos2])
                sync(e + 1, myrow)

            do3 = (maxkey - T) >= (1 << 20)
            sort_pass(0, AK, AI, BK, BI, E_S0, myrow)
            sort_pass(1, BK, BI, AK, AI, E_S0 + 2, myrow)

            @pl.when(do3)
            def _p2():
                sort_pass(2, AK, AI, BK, BI, E_S0 + 4, myrow)

            sb = pl.multiple_of(t * SLICE, SLICE)
            fk = pl.multiple_of(jnp.where(do3, BK, AK) + sb, 128)
            fi = pl.multiple_of(jnp.where(do3, BI, AI) + sb, 128)
            pltpu.sync_copy(sp_all.at[pl.ds(fk, SLICE)], v_skey)
            pltpu.sync_copy(sp_all.at[pl.ds(fi, SLICE)], v_sidx)

            def clampi(i, _):
                off = pl.multiple_of(i * 16, 16)
                v_sidx[pl.ds(off, 16)] = jnp.clip(v_sidx[pl.ds(off, 16)],
                                                  0, N - 1)
                return _
            lax.fori_loop(0, SLV, clampi, 0)
            pltpu.async_copy(joffx_hbm.at[v_sidx], v_jx, sem).wait()
            pltpu.async_copy(joffy_hbm.at[v_sidx], v_jy, sem).wait()

            def emit(i, _):
                off = pl.multiple_of(i * 16, 16)
                kk = v_skey[pl.ds(off, 16)]
                idx = v_sidx[pl.ds(off, 16)]
                score = plsc.bitcast(kk + T, jnp.float32)
                rowf = lax.shift_right_logical(idx, 9).astype(jnp.float32)
                colf = jnp.bitwise_and(idx, W - 1).astype(jnp.float32)
                v_jx[pl.ds(off, 16)] = (colf + v_jx[pl.ds(off, 16)]) + 0.5
                v_jy[pl.ds(off, 16)] = (rowf + v_jy[pl.ds(off, 16)]) + 0.5
                v_os[pl.ds(off, 16)] = score
                return _
            lax.fori_loop(0, SLV, emit, 0)
            pltpu.sync_copy(v_jx, outx_hbm.at[pl.ds(sb, SLICE)])
            pltpu.sync_copy(v_jy, outy_hbm.at[pl.ds(sb, SLICE)])
            pltpu.sync_copy(v_os, outs_hbm.at[pl.ds(sb, SLICE)])

            myrow = sync(E_LAST, myrow, last=True)
            v_tagbank[pl.ds(31 * 16, 16)] = _splat(PARK)
            for tt in range(NT):
                @pl.when(t == tt)
                def _(tt=tt):
                    pltpu.sync_copy(v_tagbank.at[pl.ds(31 * 16, 16)],
                                    sp_all.at[pl.ds(TAGS + tt * 16, 16)])

    return sc_kernel


_SC_KERNEL = None


def kernel(jloc, joff, k):
    global _SC_KERNEL
    if _SC_KERNEL is None:
        _SC_KERNEL = _sc_topk_kernel()
    keys2d = pl.pallas_call(
        _nms_body,
        out_shape=jax.ShapeDtypeStruct((H, W), jnp.int32),
    )(jloc)
    keys = keys2d.reshape(-1)
    joff_flat = joff.reshape(2, -1)
    outx, outy, outs_ = _SC_KERNEL(keys, joff_flat[0], joff_flat[1])
    x = outx[:K]
    y = outy[:K]
    scores = outs_[:K] + (jnp.asarray(k) - K).astype(jnp.float32)
    junctions = jnp.stack((x, y)).T
    return jnp.concatenate([junctions, scores[:, None]], axis=1)


# speedup vs baseline: 1.9771x; 1.0002x over previous
import functools

import jax
import jax.numpy as jnp
from jax import lax
from jax.experimental import pallas as pl
from jax.experimental.pallas import tpu as pltpu
from jax.experimental.pallas import tpu_sc as plsc

H = W = 512
N = H * W
NT = 16
CH = N // NT
CHV = CH // 16
K = 20000
KPAD = 20480
SLICE = KPAD // NT
SLV = SLICE // 16
TRASH = KPAD
M = KPAD + 2048
MAGIC = 0x51C0000
PARK = MAGIC + 99
ASHIFT = 18
NB = 1 << (30 - ASHIFT)
NBS = NB // NT
LOWM = (1 << ASHIFT) - 1

TAGS = 0
HIST = 256
AK = HIST + NT * NB
AI = AK + M
BK = AI + M
BI = BK + M
SP_WORDS = BI + M

E_A1 = 1
E_A2 = 2
E_A3 = 3
E_B0 = 4
E_C1 = 16
E_C2 = 17
E_S0 = 18
E_F = 24
E_LAST = E_F


def _nms_body(jloc_ref, out_ref):
    x = jloc_ref[0]
    neginf = jnp.float32(-jnp.inf)
    up = jnp.concatenate([x[1:], jnp.full((1, W), neginf, x.dtype)], axis=0)
    dn = jnp.concatenate([jnp.full((1, W), neginf, x.dtype), x[:-1]], axis=0)
    m = jnp.maximum(jnp.maximum(x, up), dn)
    lf = jnp.concatenate([m[:, 1:], jnp.full((H, 1), neginf, x.dtype)], axis=1)
    rt = jnp.concatenate([jnp.full((H, 1), neginf, x.dtype), m[:, :-1]], axis=1)
    ap = jnp.maximum(jnp.maximum(m, lf), rt)
    a = jnp.where(x == ap, x, jnp.float32(0.0))
    out_ref[...] = lax.bitcast_convert_type(a, jnp.int32)


def _splat(v):
    return jnp.full((16,), v, jnp.int32)


def _sc_topk_kernel():
    mesh = plsc.VectorSubcoreMesh(core_axis_name="c", subcore_axis_name="s")

    @functools.partial(
        pl.kernel, mesh=mesh,
        out_type=[jax.ShapeDtypeStruct((KPAD,), jnp.float32),
                  jax.ShapeDtypeStruct((KPAD,), jnp.float32),
                  jax.ShapeDtypeStruct((KPAD,), jnp.float32)],
        scratch_types=[
            pltpu.VMEM((CH,), jnp.int32),
            pltpu.VMEM((CH,), jnp.int32),
            pltpu.VMEM((CH,), jnp.int32),
            pltpu.VMEM((16384,), jnp.int32),
            pltpu.VMEM((NBS,), jnp.int32),
            pltpu.VMEM((NBS,), jnp.int32),
            pltpu.VMEM((256,), jnp.int32),
            pltpu.VMEM((16,), jnp.int32),
            pltpu.VMEM((32 * 16,), jnp.int32),
            pltpu.VMEM((3 * 1024,), jnp.int32),
            pltpu.VMEM((1024,), jnp.int32),
            pltpu.VMEM((1024,), jnp.int32),
            pltpu.VMEM((SLICE,), jnp.int32),
            pltpu.VMEM((SLICE,), jnp.int32),
            pltpu.VMEM((SLICE,), jnp.int32),
            pltpu.VMEM((SLICE,), jnp.int32),
            pltpu.VMEM((SLICE,), jnp.float32),
            pltpu.VMEM((SLICE,), jnp.float32),
            pltpu.VMEM((SLICE,), jnp.float32),
            pltpu.VMEM_SHARED((SP_WORDS,), jnp.int32),
            pltpu.SemaphoreType.DMA,
        ],
        compiler_params=pltpu.CompilerParams(needs_layout_passes=False),
    )
    def sc_kernel(keys_hbm, joffx_hbm, joffy_hbm, outx_hbm, outy_hbm,
                  outs_hbm, v_chunk, v_gidx, v_pos, v_hist, v_g, v_g2,
                  v_land, v_h8, v_tagbank, v_shbank, v_bases, v_shist,
                  v_skey, v_sidx, v_spos, v_spos2, v_jx, v_jy, v_os,
                  sp_all, sem):
        cid = lax.axis_index("c")
        t = lax.axis_index("s")

        def i16():
            return lax.iota(jnp.int32, 16)

        def publish(e, myrow):
            row = jnp.where(i16() == 0, MAGIC + e, myrow)
            v_tagbank[pl.ds(e * 16, 16)] = row
            for tt in range(NT):
                @pl.when(t == tt)
                def _(tt=tt, e=e):
                    pltpu.sync_copy(v_tagbank.at[pl.ds(e * 16, 16)],
                                    sp_all.at[pl.ds(TAGS + tt * 16, 16)])
            return row

        def spin(e, last=False):
            def cond(carry):
                notdone, it = carry
                return jnp.logical_and(notdone, it < 300000)

            def body(carry):
                _, it = carry
                pltpu.sync_copy(sp_all.at[pl.ds(TAGS, 256)], v_land)
                col = plsc.load_gather(v_land, [i16() * 16])
                if last:
                    ok_lane = col >= MAGIC + e
                else:
                    ok_lane = jnp.logical_and(col >= MAGIC + e,
                                              col <= MAGIC + 64)
                nok = jnp.sum(ok_lane.astype(jnp.int32), axis=0)
                return nok < 16, it + 1

            lax.while_loop(cond, body, (jnp.bool_(True), jnp.int32(0)))

        def sync(e, myrow, last=False):
            row = publish(e, myrow)
            spin(e, last=last)
            return row

        def colv(c):
            return plsc.load_gather(v_land, [i16() * 16 + c])

        @pl.when(cid == 0)
        def _core0():
            ones = _splat(1)
            zeros = _splat(0)
            myrow = zeros

            pltpu.sync_copy(keys_hbm.at[pl.ds(t * CH, CH)], v_chunk)

            def zh(i, _):
                off = pl.multiple_of(i * 16, 16)
                v_hist[pl.ds(off, 16)] = zeros
                return _
            lax.fori_loop(0, NB // 16, zh, 0)

            def ah(i, mx):
                for u in range(4):
                    off = pl.multiple_of(i * 64 + u * 16, 16)
                    key = v_chunk[pl.ds(off, 16)]
                    d = lax.shift_right_logical(key, ASHIFT)
                    plsc.addupdate_scatter(v_hist, [d], ones)
                    mx = jnp.maximum(mx, key)
                return mx
            maxvec = lax.fori_loop(0, CHV // 4, ah, zeros)
            maxkey_t = jnp.max(maxvec, axis=0)
            for tt in range(NT):
                @pl.when(t == tt)
                def _(tt=tt):
                    pltpu.sync_copy(v_hist.at[pl.ds(0, NB)],
                                    sp_all.at[pl.ds(HIST + tt * NB, NB)])
            myrow = sync(E_A1, myrow)

            def zg(i, _):
                off = pl.multiple_of(i * 16, 16)
                v_g[pl.ds(off, 16)] = zeros
                return _
            lax.fori_loop(0, NBS // 16, zg, 0)

            for rr in range(NT):
                pltpu.sync_copy(
                    sp_all.at[pl.ds(HIST + rr * NB + t * NBS, NBS)], v_g2)

                def addv(i, __):
                    off = pl.multiple_of(i * 16, 16)
                    v_g[pl.ds(off, 16)] = (v_g[pl.ds(off, 16)]
                                           + v_g2[pl.ds(off, 16)])
                    return __
                lax.fori_loop(0, NBS // 16, addv, 0)

            def tot_acc(i, acc):
                off = pl.multiple_of(i * 16, 16)
                return acc + v_g[pl.ds(off, 16)]
            tot_vec = lax.fori_loop(0, NBS // 16, tot_acc, zeros)
            tot_slice = jnp.sum(tot_vec, axis=0)
            myrow = jnp.where(i16() == 12, tot_slice, myrow)
            myrow = jnp.where(i16() == 15, maxkey_t, myrow)
            myrow = sync(E_A2, myrow)

            maxkey = jnp.max(colv(15), axis=0)
            tots = colv(12)
            sfx = lax.rev(plsc.cumsum(lax.rev(tots, (0,))), (0,))
            s_above_v = sfx - tots
            s_above = jnp.sum(jnp.where(i16() == t, s_above_v, 0), axis=0)

            def find(iv, carry):
                found, bstar, n_hi, running = carry
                v = NBS // 16 - 1 - iv
                off = pl.multiple_of(v * 16, 16)
                c = v_g[pl.ds(off, 16)]
                sfx_in = lax.rev(plsc.cumsum(lax.rev(c, (0,))), (0,))
                s_excl = running + sfx_in - c
                cond = jnp.logical_and(s_excl < K, K <= s_excl + c)
                anyc = jnp.sum(cond.astype(jnp.int32), axis=0) > 0
                d_here = jnp.sum(jnp.where(cond, t * NBS + off + i16(), 0),
                                 axis=0)
                nh_here = jnp.sum(jnp.where(cond, s_excl, 0), axis=0)
                hit = jnp.logical_and(anyc, jnp.logical_not(found))
                bstar = jnp.where(hit, d_here, bstar)
                n_hi = jnp.where(hit, nh_here, n_hi)
                found = jnp.logical_or(found, anyc)
                running = running + jnp.sum(c, axis=0)
                return found, bstar, n_hi, running
            found, bstar, n_hi, _ = lax.fori_loop(
                0, NBS // 16, find,
                (jnp.bool_(False), jnp.int32(0), jnp.int32(0), s_above))

            myrow = jnp.where(i16() == 9, found.astype(jnp.int32), myrow)
            myrow = jnp.where(i16() == 10, bstar, myrow)
            myrow = jnp.where(i16() == 11, n_hi, myrow)
            myrow = sync(E_A3, myrow)

            fcol = colv(9)
            bstar = jnp.sum(fcol * colv(10), axis=0)
            n_hi = jnp.sum(fcol * colv(11), axis=0)

            def compact(i, carry):
                cnt, cgt = carry
                for u in range(4):
                    off = pl.multiple_of(i * 64 + u * 16, 16)
                    key = v_chunk[pl.ds(off, 16)]
                    d = lax.shift_right_logical(key, ASHIFT)
                    m = d == bstar
                    mi = m.astype(jnp.int32)
                    ex = plsc.cumsum(mi) - mi
                    pos = cnt + ex
                    plsc.store_scatter(v_pos, [pos],
                                       jnp.bitwise_and(key, LOWM), mask=m)
                    cgt = cgt + jnp.sum((d > bstar).astype(jnp.int32),
                                        axis=0)
                    cnt = cnt + jnp.sum(mi, axis=0)
                return cnt, cgt
            neq_list, cgt_t = lax.fori_loop(0, CHV // 4, compact,
                                            (jnp.int32(0), jnp.int32(0)))
            nv_eq = lax.div(neq_list + 15, 16)

            n_above = n_hi
            prefix = jnp.int32(0)
            for r in range(6):
                sh = 15 - 3 * r
                v_h8[...] = zeros

                def hrow(i, _, sh=sh, prefix=prefix):
                    off = pl.multiple_of(i * 16, 16)
                    lw = v_pos[pl.ds(off, 16)]
                    valid = off + i16() < neq_list
                    pref_ok = lax.shift_right_logical(lw, sh + 3) == prefix
                    m = jnp.logical_and(valid, pref_ok)
                    b = jnp.bitwise_and(lax.shift_right_logical(lw, sh), 7)
                    plsc.addupdate_scatter(v_h8, [b], ones, mask=m)
                    return _
                lax.fori_loop(0, nv_eq, hrow, 0)
                h8 = v_h8[...]
                g = plsc.load_gather(v_h8, [jnp.clip(i16() - 1, 0, 15)])
                incols = jnp.logical_and(i16() >= 1, i16() <= 8)
                myrow = jnp.where(incols, g, myrow)
                myrow = sync(E_B0 + 2 * r, myrow)

                gcnt = zeros
                for c in range(8):
                    s = jnp.sum(colv(1 + c), axis=0)
                    gcnt = gcnt + jnp.where(i16() == c, s, 0)
                sfx8 = lax.rev(plsc.cumsum(lax.rev(gcnt, (0,))), (0,))
                ex8 = n_above + sfx8 - gcnt
                cond8 = jnp.logical_and(ex8 < K, K <= ex8 + gcnt)
                beta = jnp.sum(jnp.where(cond8, i16(), 0), axis=0)
                n_above = jnp.sum(jnp.where(cond8, ex8, 0), axis=0)
                prefix = prefix * 8 + beta
                myrow = sync(E_B0 + 2 * r + 1, myrow)

            lstar = prefix
            n_gt = n_above
            T = jnp.bitwise_or(lax.shift_left(bstar, ASHIFT), lstar)

            def c1(i, carry):
                ngt, neq = carry
                off = pl.multiple_of(i * 16, 16)
                lw = v_pos[pl.ds(off, 16)]
                valid = off + i16() < neq_list
                gt = jnp.logical_and(valid, lw > lstar)
                eq = jnp.logical_and(valid, lw == lstar)
                ngt = ngt + jnp.sum(gt.astype(jnp.int32), axis=0)
                neq = neq + jnp.sum(eq.astype(jnp.int32), axis=0)
                return ngt, neq
            ngt_eq, neq_t = lax.fori_loop(0, nv_eq, c1,
                                          (jnp.int32(0), jnp.int32(0)))
            ngt_t = cgt_t + ngt_eq
            myrow = jnp.where(i16() == 13, ngt_t, myrow)
            myrow = jnp.where(i16() == 14, neq_t, myrow)
            myrow = sync(E_C1, myrow)

            ngts = colv(13)
            neqs = colv(14)
            pre_gt = plsc.cumsum(ngts) - ngts
            pre_eq = plsc.cumsum(neqs) - neqs
            sel = i16() == t
            base_gt = jnp.sum(jnp.where(sel, pre_gt, 0), axis=0)
            eqpre = jnp.sum(jnp.where(sel, pre_eq, 0), axis=0)
            n_tie = K - n_gt

            def c2(i, carry):
                gtrun, eqrun = carry
                for u in range(4):
                    off = pl.multiple_of(i * 64 + u * 16, 16)
                    key = v_chunk[pl.ds(off, 16)]
                    m_gt = key > T
                    m_eq = key == T
                    gi = m_gt.astype(jnp.int32)
                    ei = m_eq.astype(jnp.int32)
                    ex_gt = plsc.cumsum(gi) - gi
                    ex_eq = plsc.cumsum(ei) - ei
                    pos_gt = gtrun + ex_gt
                    eq_rank = eqrun + ex_eq
                    keep_eq = jnp.logical_and(m_eq, eq_rank < n_tie)
                    pos_eq = n_gt + eq_rank
                    trash = TRASH + jnp.bitwise_and(off, 2047) + i16()
                    pos = jnp.where(m_gt, pos_gt,
                                    jnp.where(keep_eq, pos_eq, trash))
                    pos = jnp.clip(pos, 0, M - 1)
                    v_pos[pl.ds(off, 16)] = AK + pos
                    v_hist[pl.ds(off, 16)] = AI + pos
                    v_chunk[pl.ds(off, 16)] = key - T
                    v_gidx[pl.ds(off, 16)] = t * CH + off + i16()
                    gtrun = gtrun + jnp.sum(gi, axis=0)
                    eqrun = eqrun + jnp.sum(ei, axis=0)
                return gtrun, eqrun
            lax.fori_loop(0, CHV // 4, c2, (base_gt, eqpre))
            pltpu.sync_copy(v_chunk, sp_all.at[v_pos])
            pltpu.sync_copy(v_gidx, sp_all.at[v_hist])

            @pl.when(t == NT - 1)
            def _pads():
                def zp(i, _):
                    off = pl.multiple_of(i * 16, 16)
                    v_skey[pl.ds(off, 16)] = zeros
                    return _
                lax.fori_loop(0, (KPAD - K) // 16, zp, 0)
                pltpu.sync_copy(v_skey.at[pl.ds(0, KPAD - K)],
                                sp_all.at[pl.ds(AK + K, KPAD - K)])
                pltpu.sync_copy(v_skey.at[pl.ds(0, KPAD - K)],
                                sp_all.at[pl.ds(AI + K, KPAD - K)])
            myrow = sync(E_C2, myrow)

            def sort_pass(p, src_k, src_i, dst_k, dst_i, e, myrow):
                sb = pl.multiple_of(t * SLICE, SLICE)
                pltpu.sync_copy(sp_all.at[pl.ds(src_k + sb, SLICE)], v_skey)
                pltpu.sync_copy(sp_all.at[pl.ds(src_i + sb, SLICE)], v_sidx)

                hrow_ref = v_shbank.at[pl.ds(p * 1024, 1024)]

                def zsh(i, _):
                    off = pl.multiple_of(i * 16, 16)
                    hrow_ref[pl.ds(off, 16)] = zeros
                    return _
                lax.fori_loop(0, 64, zsh, 0)

                def shl(i, _):
                    off = pl.multiple_of(i * 16, 16)
                    kk = v_skey[pl.ds(off, 16)]
                    d = jnp.bitwise_and(
                        lax.shift_right_logical(kk, 10 * p), 1023)
                    plsc.addupdate_scatter(hrow_ref, [d], ones)
                    return _
                lax.fori_loop(0, SLV, shl, 0)
                for tt in range(NT):
                    @pl.when(t == tt)
                    def _(tt=tt):
                        pltpu.sync_copy(
                            hrow_ref,
                            sp_all.at[pl.ds(HIST + tt * NB, 1024)])
                myrow = sync(e, myrow)

                for rr in range(NT):
                    pltpu.sync_copy(sp_all.at[pl.ds(HIST + rr * NB, 1024)],
                                    v_hist.at[pl.ds(rr * 1024, 1024)])

                def dig2(i, _):
                    off = pl.multiple_of(i * 16, 16)

                    def rows(r, carry):
                        tot, pre = carry
                        roff = pl.multiple_of(r * 1024, 1024)
                        c = v_hist[pl.ds(roff + off, 16)]
                        pre = pre + jnp.where(r < t, c, 0)
                        return tot + c, pre
                    tot, pre = lax.fori_loop(0, 16, rows, (zeros, zeros))
                    v_bases[pl.ds(off, 16)] = pre
                    v_shist[pl.ds(off, 16)] = tot
                    return _
                lax.fori_loop(0, 64, dig2, 0)

                def sfxl(iv, running):
                    v = 63 - iv
                    off = pl.multiple_of(v * 16, 16)
                    c = v_shist[pl.ds(off, 16)]
                    sfx_in = lax.rev(plsc.cumsum(lax.rev(c, (0,))), (0,))
                    s_excl = running + sfx_in - c
                    v_bases[pl.ds(off, 16)] = v_bases[pl.ds(off, 16)] + s_excl
                    return running + jnp.sum(c, axis=0)
                lax.fori_loop(0, 64, sfxl, jnp.int32(0))

                def scat(i, _):
                    off = pl.multiple_of(i * 16, 16)
                    kk = v_skey[pl.ds(off, 16)]
                    d = jnp.bitwise_and(
                        lax.shift_right_logical(kk, 10 * p), 1023)
                    occ, lastm = plsc.scan_count(d)
                    base_d = plsc.load_gather(v_bases, [d])
                    pp = jnp.clip(base_d + occ - 1, 0, M - 1)
                    v_spos[pl.ds(off, 16)] = dst_k + pp
                    v_spos2[pl.ds(off, 16)] = dst_i + pp
                    plsc.addupdate_scatter(v_bases, [d], occ, mask=lastm)
                    return _
                lax.fori_loop(0, SLV, scat, 0)
                pltpu.sync_copy(v_skey, sp_all.at[v_spos])
                pltpu.sync_copy(v_sidx, sp_all.at[v_spos2])
                sync(e + 1, myrow)

            do3 = (maxkey - T) >= (1 << 20)
            sort_pass(0, AK, AI, BK, BI, E_S0, myrow)
            sort_pass(1, BK, BI, AK, AI, E_S0 + 2, myrow)

            @pl.when(do3)
            def _p2():
                sort_pass(2, AK, AI, BK, BI, E_S0 + 4, myrow)

            sb = pl.multiple_of(t * SLICE, SLICE)
            fk = pl.multiple_of(jnp.where(do3, BK, AK) + sb, 128)
            fi = pl.multiple_of(jnp.where(do3, BI, AI) + sb, 128)
            pltpu.sync_copy(sp_all.at[pl.ds(fk, SLICE)], v_skey)
            pltpu.sync_copy(sp_all.at[pl.ds(fi, SLICE)], v_sidx)

            def clampi(i, _):
                off = pl.multiple_of(i * 16, 16)
                v_sidx[pl.ds(off, 16)] = jnp.clip(v_sidx[pl.ds(off, 16)],
                                                  0, N - 1)
                return _
            lax.fori_loop(0, SLV, clampi, 0)
            pltpu.async_copy(joffx_hbm.at[v_sidx], v_jx, sem).wait()
            pltpu.async_copy(joffy_hbm.at[v_sidx], v_jy, sem).wait()

            def emit(i, _):
                off = pl.multiple_of(i * 16, 16)
                kk = v_skey[pl.ds(off, 16)]
                idx = v_sidx[pl.ds(off, 16)]
                score = plsc.bitcast(kk + T, jnp.float32)
                rowf = lax.shift_right_logical(idx, 9).astype(jnp.float32)
                colf = jnp.bitwise_and(idx, W - 1).astype(jnp.float32)
                v_jx[pl.ds(off, 16)] = (colf + v_jx[pl.ds(off, 16)]) + 0.5
                v_jy[pl.ds(off, 16)] = (rowf + v_jy[pl.ds(off, 16)]) + 0.5
                v_os[pl.ds(off, 16)] = score
                return _
            lax.fori_loop(0, SLV, emit, 0)
            pltpu.sync_copy(v_jx, outx_hbm.at[pl.ds(sb, SLICE)])
            pltpu.sync_copy(v_jy, outy_hbm.at[pl.ds(sb, SLICE)])
            pltpu.sync_copy(v_os, outs_hbm.at[pl.ds(sb, SLICE)])

            myrow = sync(E_LAST, myrow, last=True)
            v_tagbank[pl.ds(31 * 16, 16)] = _splat(PARK)
            for tt in range(NT):
                @pl.when(t == tt)
                def _(tt=tt):
                    pltpu.sync_copy(v_tagbank.at[pl.ds(31 * 16, 16)],
                                    sp_all.at[pl.ds(TAGS + tt * 16, 16)])

    return sc_kernel


_SC_KERNEL = None


def kernel(jloc, joff, k):
    global _SC_KERNEL
    if _SC_KERNEL is None:
        _SC_KERNEL = _sc_topk_kernel()
    keys2d = pl.pallas_call(
        _nms_body,
        out_shape=jax.ShapeDtypeStruct((H, W), jnp.int32),
    )(jloc)
    keys = keys2d.reshape(-1)
    joff_flat = joff.reshape(2, -1)
    outx, outy, outs_ = _SC_KERNEL(keys, joff_flat[0], joff_flat[1])
    x = outx[:K]
    y = outy[:K]
    scores = outs_[:K] + (jnp.asarray(k) - K).astype(jnp.float32)
    junctions = jnp.stack((x, y)).T
    return jnp.concatenate([junctions, scores[:, None]], axis=1)


# speedup vs baseline: 1.9931x; 1.0081x over previous
import functools

import jax
import jax.numpy as jnp
from jax import lax
from jax.experimental import pallas as pl
from jax.experimental.pallas import tpu as pltpu
from jax.experimental.pallas import tpu_sc as plsc

H = W = 512
N = H * W
NT = 16
CH = N // NT
CHV = CH // 16
K = 20000
KPAD = 20480
SLICE = KPAD // NT
SLV = SLICE // 16
TRASH = KPAD
M = KPAD + 2048
MAGIC = 0x51C0000
PARK = MAGIC + 99
ASHIFT = 18
NB = 1 << (30 - ASHIFT)
NBS = NB // NT
LOWM = (1 << ASHIFT) - 1

TAGS = 0
HIST = 256
AK = HIST + NT * NB
AI = AK + M
BK = AI + M
BI = BK + M
SP_WORDS = BI + M

E_A1 = 1
E_A2 = 2
E_A3 = 3
E_B0 = 4
E_C1 = 10
E_C2 = 11
E_S0 = 12
E_F = 18
E_LAST = E_F


def _nms_body(jloc_ref, out_ref):
    x = jloc_ref[0]
    neginf = jnp.float32(-jnp.inf)
    up = jnp.concatenate([x[1:], jnp.full((1, W), neginf, x.dtype)], axis=0)
    dn = jnp.concatenate([jnp.full((1, W), neginf, x.dtype), x[:-1]], axis=0)
    m = jnp.maximum(jnp.maximum(x, up), dn)
    lf = jnp.concatenate([m[:, 1:], jnp.full((H, 1), neginf, x.dtype)], axis=1)
    rt = jnp.concatenate([jnp.full((H, 1), neginf, x.dtype), m[:, :-1]], axis=1)
    ap = jnp.maximum(jnp.maximum(m, lf), rt)
    a = jnp.where(x == ap, x, jnp.float32(0.0))
    out_ref[...] = lax.bitcast_convert_type(a, jnp.int32)


def _splat(v):
    return jnp.full((16,), v, jnp.int32)


def _sc_topk_kernel():
    mesh = plsc.VectorSubcoreMesh(core_axis_name="c", subcore_axis_name="s")

    @functools.partial(
        pl.kernel, mesh=mesh,
        out_type=[jax.ShapeDtypeStruct((KPAD,), jnp.float32),
                  jax.ShapeDtypeStruct((KPAD,), jnp.float32),
                  jax.ShapeDtypeStruct((KPAD,), jnp.float32)],
        scratch_types=[
            pltpu.VMEM((CH,), jnp.int32),
            pltpu.VMEM((CH,), jnp.int32),
            pltpu.VMEM((CH,), jnp.int32),
            pltpu.VMEM((16384,), jnp.int32),
            pltpu.VMEM((NBS,), jnp.int32),
            pltpu.VMEM((NBS,), jnp.int32),
            pltpu.VMEM((256,), jnp.int32),
            pltpu.VMEM((128,), jnp.int32),
            pltpu.VMEM((16,), jnp.int32),
            pltpu.VMEM((32 * 16,), jnp.int32),
            pltpu.VMEM((3 * 1024,), jnp.int32),
            pltpu.VMEM((1024,), jnp.int32),
            pltpu.VMEM((1024,), jnp.int32),
            pltpu.VMEM((SLICE,), jnp.int32),
            pltpu.VMEM((SLICE,), jnp.int32),
            pltpu.VMEM((SLICE,), jnp.int32),
            pltpu.VMEM((SLICE,), jnp.int32),
            pltpu.VMEM((SLICE,), jnp.float32),
            pltpu.VMEM((SLICE,), jnp.float32),
            pltpu.VMEM((SLICE,), jnp.float32),
            pltpu.VMEM_SHARED((SP_WORDS,), jnp.int32),
            pltpu.SemaphoreType.DMA,
        ],
        compiler_params=pltpu.CompilerParams(needs_layout_passes=False),
    )
    def sc_kernel(keys_hbm, joffx_hbm, joffy_hbm, outx_hbm, outy_hbm,
                  outs_hbm, v_chunk, v_gidx, v_pos, v_hist, v_g, v_g2,
                  v_land, v_landB, v_h8, v_tagbank, v_shbank, v_bases, v_shist,
                  v_skey, v_sidx, v_spos, v_spos2, v_jx, v_jy, v_os,
                  sp_all, sem):
        cid = lax.axis_index("c")
        t = lax.axis_index("s")

        def i16():
            return lax.iota(jnp.int32, 16)

        def publish(e, myrow):
            row = jnp.where(i16() == 0, MAGIC + e, myrow)
            v_tagbank[pl.ds(e * 16, 16)] = row
            for tt in range(NT):
                @pl.when(t == tt)
                def _(tt=tt, e=e):
                    pltpu.sync_copy(v_tagbank.at[pl.ds(e * 16, 16)],
                                    sp_all.at[pl.ds(TAGS + tt * 16, 16)])
            return row

        def spin(e, last=False):
            def cond(carry):
                notdone, it = carry
                return jnp.logical_and(notdone, it < 300000)

            def body(carry):
                _, it = carry
                pltpu.sync_copy(sp_all.at[pl.ds(TAGS, 256)], v_land)
                col = plsc.load_gather(v_land, [i16() * 16])
                if last:
                    ok_lane = col >= MAGIC + e
                else:
                    ok_lane = jnp.logical_and(col >= MAGIC + e,
                                              col <= MAGIC + 64)
                nok = jnp.sum(ok_lane.astype(jnp.int32), axis=0)
                return nok < 16, it + 1

            lax.while_loop(cond, body, (jnp.bool_(True), jnp.int32(0)))

        def sync(e, myrow, last=False):
            row = publish(e, myrow)
            spin(e, last=last)
            return row

        def colv(c):
            return plsc.load_gather(v_land, [i16() * 16 + c])

        @pl.when(cid == 0)
        def _core0():
            ones = _splat(1)
            zeros = _splat(0)
            myrow = zeros

            pltpu.sync_copy(keys_hbm.at[pl.ds(t * CH, CH)], v_chunk)

            def zh(i, _):
                off = pl.multiple_of(i * 16, 16)
                v_hist[pl.ds(off, 16)] = zeros
                return _
            lax.fori_loop(0, NB // 16, zh, 0)

            def ah(i, mx):
                for u in range(4):
                    off = pl.multiple_of(i * 64 + u * 16, 16)
                    key = v_chunk[pl.ds(off, 16)]
                    d = lax.shift_right_logical(key, ASHIFT)
                    plsc.addupdate_scatter(v_hist, [d], ones)
                    mx = jnp.maximum(mx, key)
                return mx
            maxvec = lax.fori_loop(0, CHV // 4, ah, zeros)
            maxkey_t = jnp.max(maxvec, axis=0)
            for tt in range(NT):
                @pl.when(t == tt)
                def _(tt=tt):
                    pltpu.sync_copy(v_hist.at[pl.ds(0, NB)],
                                    sp_all.at[pl.ds(HIST + tt * NB, NB)])
            myrow = sync(E_A1, myrow)

            def zg(i, _):
                off = pl.multiple_of(i * 16, 16)
                v_g[pl.ds(off, 16)] = zeros
                return _
            lax.fori_loop(0, NBS // 16, zg, 0)

            for rr in range(NT):
                pltpu.sync_copy(
                    sp_all.at[pl.ds(HIST + rr * NB + t * NBS, NBS)], v_g2)

                def addv(i, __):
                    off = pl.multiple_of(i * 16, 16)
                    v_g[pl.ds(off, 16)] = (v_g[pl.ds(off, 16)]
                                           + v_g2[pl.ds(off, 16)])
                    return __
                lax.fori_loop(0, NBS // 16, addv, 0)

            def tot_acc(i, acc):
                off = pl.multiple_of(i * 16, 16)
                return acc + v_g[pl.ds(off, 16)]
            tot_vec = lax.fori_loop(0, NBS // 16, tot_acc, zeros)
            tot_slice = jnp.sum(tot_vec, axis=0)
            myrow = jnp.where(i16() == 12, tot_slice, myrow)
            myrow = jnp.where(i16() == 15, maxkey_t, myrow)
            myrow = sync(E_A2, myrow)

            maxkey = jnp.max(colv(15), axis=0)
            tots = colv(12)
            sfx = lax.rev(plsc.cumsum(lax.rev(tots, (0,))), (0,))
            s_above_v = sfx - tots
            s_above = jnp.sum(jnp.where(i16() == t, s_above_v, 0), axis=0)

            def find(iv, carry):
                found, bstar, n_hi, running = carry
                v = NBS // 16 - 1 - iv
                off = pl.multiple_of(v * 16, 16)
                c = v_g[pl.ds(off, 16)]
                sfx_in = lax.rev(plsc.cumsum(lax.rev(c, (0,))), (0,))
                s_excl = running + sfx_in - c
                cond = jnp.logical_and(s_excl < K, K <= s_excl + c)
                anyc = jnp.sum(cond.astype(jnp.int32), axis=0) > 0
                d_here = jnp.sum(jnp.where(cond, t * NBS + off + i16(), 0),
                                 axis=0)
                nh_here = jnp.sum(jnp.where(cond, s_excl, 0), axis=0)
                hit = jnp.logical_and(anyc, jnp.logical_not(found))
                bstar = jnp.where(hit, d_here, bstar)
                n_hi = jnp.where(hit, nh_here, n_hi)
                found = jnp.logical_or(found, anyc)
                running = running + jnp.sum(c, axis=0)
                return found, bstar, n_hi, running
            found, bstar, n_hi, _ = lax.fori_loop(
                0, NBS // 16, find,
                (jnp.bool_(False), jnp.int32(0), jnp.int32(0), s_above))

            myrow = jnp.where(i16() == 9, found.astype(jnp.int32), myrow)
            myrow = jnp.where(i16() == 10, bstar, myrow)
            myrow = jnp.where(i16() == 11, n_hi, myrow)
            myrow = sync(E_A3, myrow)

            fcol = colv(9)
            bstar = jnp.sum(fcol * colv(10), axis=0)
            n_hi = jnp.sum(fcol * colv(11), axis=0)

            def compact(i, carry):
                cnt, cgt = carry
                for u in range(4):
                    off = pl.multiple_of(i * 64 + u * 16, 16)
                    key = v_chunk[pl.ds(off, 16)]
                    d = lax.shift_right_logical(key, ASHIFT)
                    m = d == bstar
                    mi = m.astype(jnp.int32)
                    ex = plsc.cumsum(mi) - mi
                    pos = cnt + ex
                    plsc.store_scatter(v_pos, [pos],
                                       jnp.bitwise_and(key, LOWM), mask=m)
                    cgt = cgt + jnp.sum((d > bstar).astype(jnp.int32),
                                        axis=0)
                    cnt = cnt + jnp.sum(mi, axis=0)
                return cnt, cgt
            neq_list, cgt_t = lax.fori_loop(0, CHV // 4, compact,
                                            (jnp.int32(0), jnp.int32(0)))
            nv_eq = lax.div(neq_list + 15, 16)

            n_above = n_hi
            prefix = jnp.int32(0)
            for r in range(6):
                sh = 15 - 3 * r
                v_h8[...] = zeros

                def hrow(i, _, sh=sh, prefix=prefix):
                    off = pl.multiple_of(i * 16, 16)
                    lw = v_pos[pl.ds(off, 16)]
                    valid = off + i16() < neq_list
                    pref_ok = lax.shift_right_logical(lw, sh + 3) == prefix
                    m = jnp.logical_and(valid, pref_ok)
                    b = jnp.bitwise_and(lax.shift_right_logical(lw, sh), 7)
                    plsc.addupdate_scatter(v_h8, [b], ones, mask=m)
                    return _
                lax.fori_loop(0, nv_eq, hrow, 0)
                v_tagbank[pl.ds((24 + r) * 16, 16)] = v_h8[...]
                for tt in range(NT):
                    @pl.when(t == tt)
                    def _(tt=tt, r=r):
                        pltpu.sync_copy(
                            v_tagbank.at[pl.ds((24 + r) * 16, 8)],
                            sp_all.at[pl.ds(HIST + 128 * r + 8 * tt, 8)])
                myrow = sync(E_B0 + r, myrow)
                pltpu.sync_copy(sp_all.at[pl.ds(HIST + 128 * r, 128)],
                                v_landB)
                gcnt = zeros
                for tt in range(NT):
                    g = plsc.load_gather(
                        v_landB, [8 * tt + jnp.bitwise_and(i16(), 7)])
                    gcnt = gcnt + jnp.where(i16() < 8, g, 0)
                sfx8 = lax.rev(plsc.cumsum(lax.rev(gcnt, (0,))), (0,))
                ex8 = n_above + sfx8 - gcnt
                cond8 = jnp.logical_and(ex8 < K, K <= ex8 + gcnt)
                beta = jnp.sum(jnp.where(cond8, i16(), 0), axis=0)
                n_above = jnp.sum(jnp.where(cond8, ex8, 0), axis=0)
                prefix = prefix * 8 + beta

            lstar = prefix
            n_gt = n_above
            T = jnp.bitwise_or(lax.shift_left(bstar, ASHIFT), lstar)

            def c1(i, carry):
                ngt, neq = carry
                off = pl.multiple_of(i * 16, 16)
                lw = v_pos[pl.ds(off, 16)]
                valid = off + i16() < neq_list
                gt = jnp.logical_and(valid, lw > lstar)
                eq = jnp.logical_and(valid, lw == lstar)
                ngt = ngt + jnp.sum(gt.astype(jnp.int32), axis=0)
                neq = neq + jnp.sum(eq.astype(jnp.int32), axis=0)
                return ngt, neq
            ngt_eq, neq_t = lax.fori_loop(0, nv_eq, c1,
                                          (jnp.int32(0), jnp.int32(0)))
            ngt_t = cgt_t + ngt_eq
            myrow = jnp.where(i16() == 13, ngt_t, myrow)
            myrow = jnp.where(i16() == 14, neq_t, myrow)
            myrow = sync(E_C1, myrow)

            ngts = colv(13)
            neqs = colv(14)
            pre_gt = plsc.cumsum(ngts) - ngts
            pre_eq = plsc.cumsum(neqs) - neqs
            sel = i16() == t
            base_gt = jnp.sum(jnp.where(sel, pre_gt, 0), axis=0)
            eqpre = jnp.sum(jnp.where(sel, pre_eq, 0), axis=0)
            n_tie = K - n_gt

            def c2(i, carry):
                gtrun, eqrun = carry
                for u in range(4):
                    off = pl.multiple_of(i * 64 + u * 16, 16)
                    key = v_chunk[pl.ds(off, 16)]
                    m_gt = key > T
                    m_eq = key == T
                    gi = m_gt.astype(jnp.int32)
                    ei = m_eq.astype(jnp.int32)
                    ex_gt = plsc.cumsum(gi) - gi
                    ex_eq = plsc.cumsum(ei) - ei
                    pos_gt = gtrun + ex_gt
                    eq_rank = eqrun + ex_eq
                    keep_eq = jnp.logical_and(m_eq, eq_rank < n_tie)
                    pos_eq = n_gt + eq_rank
                    trash = TRASH + jnp.bitwise_and(off, 2047) + i16()
                    pos = jnp.where(m_gt, pos_gt,
                                    jnp.where(keep_eq, pos_eq, trash))
                    pos = jnp.clip(pos, 0, M - 1)
                    v_pos[pl.ds(off, 16)] = AK + pos
                    v_hist[pl.ds(off, 16)] = AI + pos
                    v_chunk[pl.ds(off, 16)] = key - T
                    v_gidx[pl.ds(off, 16)] = t * CH + off + i16()
                    gtrun = gtrun + jnp.sum(gi, axis=0)
                    eqrun = eqrun + jnp.sum(ei, axis=0)
                return gtrun, eqrun
            lax.fori_loop(0, CHV // 4, c2, (base_gt, eqpre))
            pltpu.sync_copy(v_chunk, sp_all.at[v_pos])
            pltpu.sync_copy(v_gidx, sp_all.at[v_hist])

            @pl.when(t == NT - 1)
            def _pads():
                def zp(i, _):
                    off = pl.multiple_of(i * 16, 16)
                    v_skey[pl.ds(off, 16)] = zeros
                    return _
                lax.fori_loop(0, (KPAD - K) // 16, zp, 0)
                pltpu.sync_copy(v_skey.at[pl.ds(0, KPAD - K)],
                                sp_all.at[pl.ds(AK + K, KPAD - K)])
                pltpu.sync_copy(v_skey.at[pl.ds(0, KPAD - K)],
                                sp_all.at[pl.ds(AI + K, KPAD - K)])
            myrow = sync(E_C2, myrow)

            def sort_pass(p, src_k, src_i, dst_k, dst_i, e, myrow):
                sb = pl.multiple_of(t * SLICE, SLICE)
                pltpu.sync_copy(sp_all.at[pl.ds(src_k + sb, SLICE)], v_skey)
                pltpu.sync_copy(sp_all.at[pl.ds(src_i + sb, SLICE)], v_sidx)

                hrow_ref = v_shbank.at[pl.ds(p * 1024, 1024)]

                def zsh(i, _):
                    off = pl.multiple_of(i * 16, 16)
                    hrow_ref[pl.ds(off, 16)] = zeros
                    return _
                lax.fori_loop(0, 64, zsh, 0)

                def shl(i, _):
                    off = pl.multiple_of(i * 16, 16)
                    kk = v_skey[pl.ds(off, 16)]
                    d = jnp.bitwise_and(
                        lax.shift_right_logical(kk, 10 * p), 1023)
                    plsc.addupdate_scatter(hrow_ref, [d], ones)
                    return _
                lax.fori_loop(0, SLV, shl, 0)
                for tt in range(NT):
                    @pl.when(t == tt)
                    def _(tt=tt):
                        pltpu.sync_copy(
                            hrow_ref,
                            sp_all.at[pl.ds(HIST + tt * NB, 1024)])
                myrow = sync(e, myrow)

                for rr in range(NT):
                    pltpu.sync_copy(sp_all.at[pl.ds(HIST + rr * NB, 1024)],
                                    v_hist.at[pl.ds(rr * 1024, 1024)])

                def dig2(i, _):
                    off = pl.multiple_of(i * 16, 16)

                    def rows(r, carry):
                        tot, pre = carry
                        roff = pl.multiple_of(r * 1024, 1024)
                        c = v_hist[pl.ds(roff + off, 16)]
                        pre = pre + jnp.where(r < t, c, 0)
                        return tot + c, pre
                    tot, pre = lax.fori_loop(0, 16, rows, (zeros, zeros))
                    v_bases[pl.ds(off, 16)] = pre
                    v_shist[pl.ds(off, 16)] = tot
                    return _
                lax.fori_loop(0, 64, dig2, 0)

                def sfxl(iv, running):
                    v = 63 - iv
                    off = pl.multiple_of(v * 16, 16)
                    c = v_shist[pl.ds(off, 16)]
                    sfx_in = lax.rev(plsc.cumsum(lax.rev(c, (0,))), (0,))
                    s_excl = running + sfx_in - c
                    v_bases[pl.ds(off, 16)] = v_bases[pl.ds(off, 16)] + s_excl
                    return running + jnp.sum(c, axis=0)
                lax.fori_loop(0, 64, sfxl, jnp.int32(0))

                def scat(i, _):
                    off = pl.multiple_of(i * 16, 16)
                    kk = v_skey[pl.ds(off, 16)]
                    d = jnp.bitwise_and(
                        lax.shift_right_logical(kk, 10 * p), 1023)
                    occ, lastm = plsc.scan_count(d)
                    base_d = plsc.load_gather(v_bases, [d])
                    pp = jnp.clip(base_d + occ - 1, 0, M - 1)
                    v_spos[pl.ds(off, 16)] = dst_k + pp
                    v_spos2[pl.ds(off, 16)] = dst_i + pp
                    plsc.addupdate_scatter(v_bases, [d], occ, mask=lastm)
                    return _
                lax.fori_loop(0, SLV, scat, 0)
                pltpu.sync_copy(v_skey, sp_all.at[v_spos])
                pltpu.sync_copy(v_sidx, sp_all.at[v_spos2])
                sync(e + 1, myrow)

            do3 = (maxkey - T) >= (1 << 20)
            sort_pass(0, AK, AI, BK, BI, E_S0, myrow)
            sort_pass(1, BK, BI, AK, AI, E_S0 + 2, myrow)

            @pl.when(do3)
            def _p2():
                sort_pass(2, AK, AI, BK, BI, E_S0 + 4, myrow)

            sb = pl.multiple_of(t * SLICE, SLICE)
            fk = pl.multiple_of(jnp.where(do3, BK, AK) + sb, 128)
            fi = pl.multiple_of(jnp.where(do3, BI, AI) + sb, 128)
            pltpu.sync_copy(sp_all.at[pl.ds(fk, SLICE)], v_skey)
            pltpu.sync_copy(sp_all.at[pl.ds(fi, SLICE)], v_sidx)

            def clampi(i, _):
                off = pl.multiple_of(i * 16, 16)
                v_sidx[pl.ds(off, 16)] = jnp.clip(v_sidx[pl.ds(off, 16)],
                                                  0, N - 1)
                return _
            lax.fori_loop(0, SLV, clampi, 0)
            pltpu.async_copy(joffx_hbm.at[v_sidx], v_jx, sem).wait()
            pltpu.async_copy(joffy_hbm.at[v_sidx], v_jy, sem).wait()

            def emit(i, _):
                off = pl.multiple_of(i * 16, 16)
                kk = v_skey[pl.ds(off, 16)]
                idx = v_sidx[pl.ds(off, 16)]
                score = plsc.bitcast(kk + T, jnp.float32)
                rowf = lax.shift_right_logical(idx, 9).astype(jnp.float32)
                colf = jnp.bitwise_and(idx, W - 1).astype(jnp.float32)
                v_jx[pl.ds(off, 16)] = (colf + v_jx[pl.ds(off, 16)]) + 0.5
                v_jy[pl.ds(off, 16)] = (rowf + v_jy[pl.ds(off, 16)]) + 0.5
                v_os[pl.ds(off, 16)] = score
                return _
            lax.fori_loop(0, SLV, emit, 0)
            pltpu.sync_copy(v_jx, outx_hbm.at[pl.ds(sb, SLICE)])
            pltpu.sync_copy(v_jy, outy_hbm.at[pl.ds(sb, SLICE)])
            pltpu.sync_copy(v_os, outs_hbm.at[pl.ds(sb, SLICE)])

            myrow = sync(E_LAST, myrow, last=True)
            v_tagbank[pl.ds(31 * 16, 16)] = _splat(PARK)
            for tt in range(NT):
                @pl.when(t == tt)
                def _(tt=tt):
                    pltpu.sync_copy(v_tagbank.at[pl.ds(31 * 16, 16)],
                                    sp_all.at[pl.ds(TAGS + tt * 16, 16)])

    return sc_kernel


_SC_KERNEL = None


def kernel(jloc, joff, k):
    global _SC_KERNEL
    if _SC_KERNEL is None:
        _SC_KERNEL = _sc_topk_kernel()
    keys2d = pl.pallas_call(
        _nms_body,
        out_shape=jax.ShapeDtypeStruct((H, W), jnp.int32),
    )(jloc)
    keys = keys2d.reshape(-1)
    joff_flat = joff.reshape(2, -1)
    outx, outy, outs_ = _SC_KERNEL(keys, joff_flat[0], joff_flat[1])
    x = outx[:K]
    y = outy[:K]
    scores = outs_[:K] + (jnp.asarray(k) - K).astype(jnp.float32)
    junctions = jnp.stack((x, y)).T
    return jnp.concatenate([junctions, scores[:, None]], axis=1)
